# baseline (speedup 1.0000x reference)
import jax
import jax.numpy as jnp
from jax import lax
from jax.experimental import pallas as pl
from jax.experimental.pallas import tpu as pltpu
from jax.experimental.pallas import tpu_sc as plsc

_BATCH = 16384
_SEQ = 100
_OUT_DIM = 128
_NW = 32
_PER_W = _BATCH // _NW
_NB = 8
_NCOPY = _PER_W // _NB
_L = 16


def _sc_fill(emb_hbm, out_hbm, scal_v, buf_v):
    c = lax.axis_index("c")
    s = lax.axis_index("s")
    wid = s * 2 + c

    pltpu.sync_copy(emb_hbm, scal_v)
    v = scal_v[...]

    def fill(j, carry):
        for b in range(_NB):
            for k in range(_OUT_DIM // _L):
                buf_v[b, j, pl.ds(k * _L, _L)] = v
        return carry

    lax.fori_loop(0, _SEQ, fill, 0)

    base = wid * _PER_W

    def copy(i, carry):
        pltpu.sync_copy(buf_v, out_hbm.at[pl.ds(base + i * _NB, _NB)])
        return carry

    lax.fori_loop(0, _NCOPY, copy, 0)


def kernel(inputs, emb_table):
    del inputs
    emb_flat = jnp.broadcast_to(emb_table.reshape((1,)), (_L,))
    return pl.kernel(
        _sc_fill,
        out_type=jax.ShapeDtypeStruct((_BATCH, _SEQ, _OUT_DIM), jnp.float32),
        mesh=plsc.VectorSubcoreMesh(core_axis_name="c", subcore_axis_name="s"),
        scratch_types=[
            pltpu.VMEM((_L,), jnp.float32),
            pltpu.VMEM((_NB, _SEQ, _OUT_DIM), jnp.float32),
        ],
        compiler_params=pltpu.CompilerParams(use_tc_tiling_on_sc=True),
    )(emb_flat)


# speedup vs baseline: 1.0204x; 1.0204x over previous
import jax
import jax.numpy as jnp
from jax.experimental import pallas as pl
from jax.experimental.pallas import tpu as pltpu

_BATCH = 16384
_SEQ = 100
_OUT_DIM = 128
_BLOCK_B = 128


def _fill_block(emb_ref, out_ref):
    out_ref[...] = jnp.broadcast_to(emb_ref[0, 0], out_ref.shape)


def kernel(inputs, emb_table):
    del inputs
    return pl.pallas_call(
        _fill_block,
        grid=(_BATCH // _BLOCK_B,),
        in_specs=[pl.BlockSpec((1, 1), lambda i: (0, 0))],
        out_specs=pl.BlockSpec((_BLOCK_B, _SEQ, _OUT_DIM), lambda i: (i, 0, 0)),
        out_shape=jax.ShapeDtypeStruct((_BATCH, _SEQ, _OUT_DIM), jnp.float32),
        compiler_params=pltpu.CompilerParams(
            dimension_semantics=("parallel",),
        ),
    )(emb_table)


# speedup vs baseline: 3.1628x; 3.0995x over previous
import jax
import jax.numpy as jnp
from jax.experimental import pallas as pl
from jax.experimental.pallas import tpu as pltpu

_BATCH = 16384
_SEQ = 100
_OUT_DIM = 128
_BLOCK_B = 128


def _fill_block(emb_ref, out_ref):
    out_ref[...] = jnp.broadcast_to(emb_ref[0, 0], out_ref.shape)


def kernel(inputs, emb_table):
    del inputs
    out = pl.pallas_call(
        _fill_block,
        grid=(_BATCH // _BLOCK_B,),
        in_specs=[pl.BlockSpec((1, 1), lambda i: (0, 0))],
        out_specs=pl.BlockSpec((_SEQ, _BLOCK_B, _OUT_DIM), lambda i: (0, i, 0)),
        out_shape=jax.ShapeDtypeStruct((_SEQ, _BATCH, _OUT_DIM), jnp.float32),
        compiler_params=pltpu.CompilerParams(
            dimension_semantics=("parallel",),
        ),
    )(emb_table)
    return jnp.transpose(out, (1, 0, 2))
